# Initial kernel scaffold; baseline (speedup 1.0000x reference)
import jax
import jax.numpy as jnp
from jax import lax
from jax.experimental import pallas as pl
from jax.experimental.pallas import tpu as pltpu

_NUM_CORES = 2
_T_CHUNK = 16


def _round_up(v, m):
    return ((v + m - 1) // m) * m


def _fused_model_kernel(x_ref, wih0_ref, whh0_ref, bi0_ref, bhn0_ref,
                        wih1_ref, whh1_ref, bi1_ref, bhn1_ref,
                        w1_ref, b1_ref, w2_ref, b2_ref,
                        out_ref,
                        g_scr, seq0_scr, seq1_scr, h0_scr, h1_scr):
    t_chunk, bb, in_size = x_ref.shape
    hidden = whh0_ref.shape[0]

    @pl.when(pl.program_id(1) == 0)
    def _():
        h0_scr[...] = jnp.zeros_like(h0_scr)
        h1_scr[...] = jnp.zeros_like(h1_scr)

    def gru_step(whh, bhn, out_scr):
        def step(tt, h):
            hb = h.astype(jnp.bfloat16)
            gh = jnp.dot(hb, whh, preferred_element_type=jnp.float32)
            gi = g_scr[tt].astype(jnp.float32)
            r = jax.nn.sigmoid(gi[:, :hidden] + gh[:, :hidden])
            z = jax.nn.sigmoid(gi[:, hidden:2 * hidden]
                               + gh[:, hidden:2 * hidden])
            n = jnp.tanh(gi[:, 2 * hidden:] + r * (gh[:, 2 * hidden:] + bhn))
            h_new = (1.0 - z) * n + z * h
            out_scr[tt] = h_new.astype(out_scr.dtype)
            return h_new
        return step

    xb = x_ref[...].reshape(t_chunk * bb, in_size).astype(jnp.bfloat16)
    g0 = jnp.dot(xb, wih0_ref[...], preferred_element_type=jnp.float32)
    g_scr[...] = (g0 + bi0_ref[...]).reshape(g_scr.shape).astype(g_scr.dtype)

    h0_scr[...] = lax.fori_loop(
        0, t_chunk,
        gru_step(whh0_ref[...], bhn0_ref[...].astype(jnp.float32), seq0_scr),
        h0_scr[...], unroll=8)

    a0 = seq0_scr[...].reshape(t_chunk * bb, hidden)
    g1 = jnp.dot(a0, wih1_ref[...], preferred_element_type=jnp.float32)
    g_scr[...] = (g1 + bi1_ref[...]).reshape(g_scr.shape).astype(g_scr.dtype)

    h1_scr[...] = lax.fori_loop(
        0, t_chunk,
        gru_step(whh1_ref[...], bhn1_ref[...].astype(jnp.float32), seq1_scr),
        h1_scr[...], unroll=8)

    hs = jnp.maximum(seq1_scr[...].reshape(t_chunk * bb, hidden), 0)
    y = jnp.dot(hs.astype(jnp.bfloat16), w1_ref[...],
                preferred_element_type=jnp.float32) + b1_ref[...]
    y = jnp.maximum(y, 0.0).astype(jnp.bfloat16)
    y = jnp.dot(y, w2_ref[...], preferred_element_type=jnp.float32) + b2_ref[...]
    out_ref[...] = y.reshape(out_ref.shape).astype(out_ref.dtype)


def kernel(x, l0_w_ih_t, l0_w_hh_t, l0_b_i, l0_b_hn,
           l1_w_ih_t, l1_w_hh_t, l1_b_i, l1_b_hn,
           fc1_w_t, fc1_b, fc2_w_t, fc2_b):
    seq, batch, in_size = x.shape
    hidden = l0_w_hh_t.shape[0]
    hidden3 = l0_w_ih_t.shape[1]
    fc1_out = fc1_w_t.shape[1]
    num_classes = fc2_w_t.shape[1]

    t_chunk = min(_T_CHUNK, seq)
    b_pad = _round_up(batch, 8 * _NUM_CORES)
    s_pad = _round_up(seq, t_chunk)
    xp = x
    if b_pad != batch or s_pad != seq:
        xp = jnp.pad(x, ((0, s_pad - seq), (0, b_pad - batch), (0, 0)))
    bb = b_pad // _NUM_CORES

    out = pl.pallas_call(
        _fused_model_kernel,
        out_shape=jax.ShapeDtypeStruct((s_pad, b_pad, num_classes),
                                       jnp.float32),
        grid_spec=pltpu.PrefetchScalarGridSpec(
            num_scalar_prefetch=0,
            grid=(_NUM_CORES, s_pad // t_chunk),
            in_specs=[
                pl.BlockSpec((t_chunk, bb, in_size), lambda b, c: (c, b, 0)),
                pl.BlockSpec((in_size, hidden3), lambda b, c: (0, 0)),
                pl.BlockSpec((hidden, hidden3), lambda b, c: (0, 0)),
                pl.BlockSpec((1, hidden3), lambda b, c: (0, 0)),
                pl.BlockSpec((1, hidden), lambda b, c: (0, 0)),
                pl.BlockSpec((hidden, hidden3), lambda b, c: (0, 0)),
                pl.BlockSpec((hidden, hidden3), lambda b, c: (0, 0)),
                pl.BlockSpec((1, hidden3), lambda b, c: (0, 0)),
                pl.BlockSpec((1, hidden), lambda b, c: (0, 0)),
                pl.BlockSpec((hidden, fc1_out), lambda b, c: (0, 0)),
                pl.BlockSpec((1, fc1_out), lambda b, c: (0, 0)),
                pl.BlockSpec((fc1_out, num_classes), lambda b, c: (0, 0)),
                pl.BlockSpec((1, num_classes), lambda b, c: (0, 0)),
            ],
            out_specs=pl.BlockSpec((t_chunk, bb, num_classes),
                                   lambda b, c: (c, b, 0)),
            scratch_shapes=[
                pltpu.VMEM((t_chunk, bb, hidden3), jnp.bfloat16),
                pltpu.VMEM((t_chunk, bb, hidden), jnp.bfloat16),
                pltpu.VMEM((t_chunk, bb, hidden), jnp.bfloat16),
                pltpu.VMEM((bb, hidden), jnp.float32),
                pltpu.VMEM((bb, hidden), jnp.float32),
            ],
        ),
        compiler_params=pltpu.CompilerParams(
            dimension_semantics=("core_parallel", "arbitrary"),
            vmem_limit_bytes=64 * 1024 * 1024,
        ),
    )(xp, l0_w_ih_t, l0_w_hh_t, l0_b_i, l0_b_hn,
      l1_w_ih_t, l1_w_hh_t, l1_b_i, l1_b_hn,
      fc1_w_t, fc1_b, fc2_w_t, fc2_b)

    if s_pad != seq or b_pad != batch:
        out = out[:seq, :batch]
    return out


# speedup vs baseline: 1.3836x; 1.3836x over previous
import jax
import jax.numpy as jnp
from jax import lax
from jax.experimental import pallas as pl
from jax.experimental.pallas import tpu as pltpu

_T_CHUNK = 16


def _round_up(v, m):
    return ((v + m - 1) // m) * m


def _fused_model_kernel(x_ref, wih0_ref, whh0_ref, bi0_ref, bhn0_ref,
                        wih1_ref, whh1_ref, bi1_ref, bhn1_ref,
                        w1_ref, b1_ref, w2_ref, b2_ref,
                        out_ref,
                        g_scr, seq0_scr, seq1_scr, h0_scr, h1_scr):
    t_chunk, bb, in_size = x_ref.shape
    hidden = whh0_ref.shape[0]

    @pl.when(pl.program_id(0) == 0)
    def _():
        h0_scr[...] = jnp.zeros_like(h0_scr)
        h1_scr[...] = jnp.zeros_like(h1_scr)

    def gru_step(whh, bhn, out_scr):
        def step(tt, h):
            hb = h.astype(jnp.bfloat16)
            gh = jnp.dot(hb, whh, preferred_element_type=jnp.float32)
            gi = g_scr[tt].astype(jnp.float32)
            r = jax.nn.sigmoid(gi[:, :hidden] + gh[:, :hidden])
            z = jax.nn.sigmoid(gi[:, hidden:2 * hidden]
                               + gh[:, hidden:2 * hidden])
            n = jnp.tanh(gi[:, 2 * hidden:] + r * (gh[:, 2 * hidden:] + bhn))
            h_new = (1.0 - z) * n + z * h
            out_scr[tt] = h_new.astype(out_scr.dtype)
            return h_new
        return step

    xb = x_ref[...].reshape(t_chunk * bb, in_size).astype(jnp.bfloat16)
    g0 = jnp.dot(xb, wih0_ref[...], preferred_element_type=jnp.float32)
    g_scr[...] = (g0 + bi0_ref[...]).reshape(g_scr.shape).astype(g_scr.dtype)

    h0_scr[...] = lax.fori_loop(
        0, t_chunk,
        gru_step(whh0_ref[...], bhn0_ref[...].astype(jnp.float32), seq0_scr),
        h0_scr[...], unroll=8)

    a0 = seq0_scr[...].reshape(t_chunk * bb, hidden)
    g1 = jnp.dot(a0, wih1_ref[...], preferred_element_type=jnp.float32)
    g_scr[...] = (g1 + bi1_ref[...]).reshape(g_scr.shape).astype(g_scr.dtype)

    h1_scr[...] = lax.fori_loop(
        0, t_chunk,
        gru_step(whh1_ref[...], bhn1_ref[...].astype(jnp.float32), seq1_scr),
        h1_scr[...], unroll=8)

    hs = jnp.maximum(seq1_scr[...].reshape(t_chunk * bb, hidden), 0)
    y = jnp.dot(hs.astype(jnp.bfloat16), w1_ref[...],
                preferred_element_type=jnp.float32) + b1_ref[...]
    y = jnp.maximum(y, 0.0).astype(jnp.bfloat16)
    y = jnp.dot(y, w2_ref[...], preferred_element_type=jnp.float32) + b2_ref[...]
    out_ref[...] = y.reshape(out_ref.shape).astype(out_ref.dtype)


def kernel(x, l0_w_ih_t, l0_w_hh_t, l0_b_i, l0_b_hn,
           l1_w_ih_t, l1_w_hh_t, l1_b_i, l1_b_hn,
           fc1_w_t, fc1_b, fc2_w_t, fc2_b):
    seq, batch, in_size = x.shape
    hidden = l0_w_hh_t.shape[0]
    hidden3 = l0_w_ih_t.shape[1]
    fc1_out = fc1_w_t.shape[1]
    num_classes = fc2_w_t.shape[1]

    t_chunk = min(_T_CHUNK, seq)
    b_pad = _round_up(batch, 8)
    s_pad = _round_up(seq, t_chunk)
    xp = x
    if b_pad != batch or s_pad != seq:
        xp = jnp.pad(x, ((0, s_pad - seq), (0, b_pad - batch), (0, 0)))

    out = pl.pallas_call(
        _fused_model_kernel,
        out_shape=jax.ShapeDtypeStruct((s_pad, b_pad, num_classes),
                                       jnp.float32),
        grid_spec=pltpu.PrefetchScalarGridSpec(
            num_scalar_prefetch=0,
            grid=(s_pad // t_chunk,),
            in_specs=[
                pl.BlockSpec((t_chunk, b_pad, in_size), lambda c: (c, 0, 0)),
                pl.BlockSpec((in_size, hidden3), lambda c: (0, 0)),
                pl.BlockSpec((hidden, hidden3), lambda c: (0, 0)),
                pl.BlockSpec((1, hidden3), lambda c: (0, 0)),
                pl.BlockSpec((1, hidden), lambda c: (0, 0)),
                pl.BlockSpec((hidden, hidden3), lambda c: (0, 0)),
                pl.BlockSpec((hidden, hidden3), lambda c: (0, 0)),
                pl.BlockSpec((1, hidden3), lambda c: (0, 0)),
                pl.BlockSpec((1, hidden), lambda c: (0, 0)),
                pl.BlockSpec((hidden, fc1_out), lambda c: (0, 0)),
                pl.BlockSpec((1, fc1_out), lambda c: (0, 0)),
                pl.BlockSpec((fc1_out, num_classes), lambda c: (0, 0)),
                pl.BlockSpec((1, num_classes), lambda c: (0, 0)),
            ],
            out_specs=pl.BlockSpec((t_chunk, b_pad, num_classes),
                                   lambda c: (c, 0, 0)),
            scratch_shapes=[
                pltpu.VMEM((t_chunk, b_pad, hidden3), jnp.bfloat16),
                pltpu.VMEM((t_chunk, b_pad, hidden), jnp.bfloat16),
                pltpu.VMEM((t_chunk, b_pad, hidden), jnp.bfloat16),
                pltpu.VMEM((b_pad, hidden), jnp.float32),
                pltpu.VMEM((b_pad, hidden), jnp.float32),
            ],
        ),
        compiler_params=pltpu.CompilerParams(
            dimension_semantics=("arbitrary",),
            vmem_limit_bytes=64 * 1024 * 1024,
        ),
    )(xp, l0_w_ih_t, l0_w_hh_t, l0_b_i, l0_b_hn,
      l1_w_ih_t, l1_w_hh_t, l1_b_i, l1_b_hn,
      fc1_w_t, fc1_b, fc2_w_t, fc2_b)

    if s_pad != seq or b_pad != batch:
        out = out[:seq, :batch]
    return out


# speedup vs baseline: 1.5743x; 1.1378x over previous
import jax
import jax.numpy as jnp
from jax import lax
from jax.experimental import pallas as pl
from jax.experimental.pallas import tpu as pltpu

_T_CHUNK = 16


def _round_up(v, m):
    return ((v + m - 1) // m) * m


def _fused_model_kernel(x_ref, wih0_ref, whh0_ref, bi0_ref, bhn0_ref,
                        wih1_ref, whh1_ref, bi1_ref, bhn1_ref,
                        w1_ref, b1_ref, w2_ref, b2_ref,
                        out_ref,
                        g0_scr, g1_scr, seq0_scr, seq1_scr, h0_scr, h1_scr):
    t_chunk, bb, in_size = x_ref.shape
    hidden = whh0_ref.shape[0]
    c = pl.program_id(0)

    @pl.when(c == 0)
    def _():
        h0_scr[...] = jnp.zeros_like(h0_scr)

    xb = x_ref[...].reshape(t_chunk * bb, in_size).astype(jnp.bfloat16)
    g0 = jnp.dot(xb, wih0_ref[...], preferred_element_type=jnp.float32)
    g0_scr[...] = (g0 + bi0_ref[...]).reshape(g0_scr.shape).astype(g0_scr.dtype)

    whh0 = whh0_ref[...]
    whh1 = whh1_ref[...]
    bhn0 = bhn0_ref[...].astype(jnp.float32)
    bhn1 = bhn1_ref[...].astype(jnp.float32)

    def gate_math(gi, gh, bhn, h):
        r = jax.nn.sigmoid(gi[:, :hidden] + gh[:, :hidden])
        z = jax.nn.sigmoid(gi[:, hidden:2 * hidden]
                           + gh[:, hidden:2 * hidden])
        n = jnp.tanh(gi[:, 2 * hidden:] + r * (gh[:, 2 * hidden:] + bhn))
        return (1.0 - z) * n + z * h

    def dual_step(tt, carry):
        h0, h1 = carry
        gh0 = jnp.dot(h0.astype(jnp.bfloat16), whh0,
                      preferred_element_type=jnp.float32)
        gh1 = jnp.dot(h1.astype(jnp.bfloat16), whh1,
                      preferred_element_type=jnp.float32)
        gi0 = g0_scr[tt].astype(jnp.float32)
        gi1 = g1_scr[tt].astype(jnp.float32)
        h0n = gate_math(gi0, gh0, bhn0, h0)
        h1n = gate_math(gi1, gh1, bhn1, h1)
        seq0_scr[tt] = h0n.astype(seq0_scr.dtype)
        seq1_scr[tt] = h1n.astype(seq1_scr.dtype)
        return (h0n, h1n)

    h0f, h1f = lax.fori_loop(0, t_chunk, dual_step,
                             (h0_scr[...], h1_scr[...]), unroll=8)
    h0_scr[...] = h0f
    h1_scr[...] = h1f

    @pl.when(c == 0)
    def _():
        h1_scr[...] = jnp.zeros_like(h1_scr)

    a0 = seq0_scr[...].reshape(t_chunk * bb, hidden)
    g1 = jnp.dot(a0, wih1_ref[...], preferred_element_type=jnp.float32)
    g1_scr[...] = (g1 + bi1_ref[...]).reshape(g1_scr.shape).astype(g1_scr.dtype)

    hs = jnp.maximum(seq1_scr[...].reshape(t_chunk * bb, hidden), 0)
    y = jnp.dot(hs.astype(jnp.bfloat16), w1_ref[...],
                preferred_element_type=jnp.float32) + b1_ref[...]
    y = jnp.maximum(y, 0.0).astype(jnp.bfloat16)
    y = jnp.dot(y, w2_ref[...], preferred_element_type=jnp.float32) + b2_ref[...]
    out_ref[...] = y.reshape(out_ref.shape).astype(out_ref.dtype)


def kernel(x, l0_w_ih_t, l0_w_hh_t, l0_b_i, l0_b_hn,
           l1_w_ih_t, l1_w_hh_t, l1_b_i, l1_b_hn,
           fc1_w_t, fc1_b, fc2_w_t, fc2_b):
    seq, batch, in_size = x.shape
    hidden = l0_w_hh_t.shape[0]
    hidden3 = l0_w_ih_t.shape[1]
    fc1_out = fc1_w_t.shape[1]
    num_classes = fc2_w_t.shape[1]

    t_chunk = min(_T_CHUNK, seq)
    b_pad = _round_up(batch, 8)
    s_pad = _round_up(seq, t_chunk)
    xp = x
    if b_pad != batch or s_pad != seq:
        xp = jnp.pad(x, ((0, s_pad - seq), (0, b_pad - batch), (0, 0)))

    num_chunks = s_pad // t_chunk
    last = num_chunks - 1

    out = pl.pallas_call(
        _fused_model_kernel,
        out_shape=jax.ShapeDtypeStruct((s_pad, b_pad, num_classes),
                                       jnp.float32),
        grid_spec=pltpu.PrefetchScalarGridSpec(
            num_scalar_prefetch=0,
            grid=(num_chunks + 1,),
            in_specs=[
                pl.BlockSpec((t_chunk, b_pad, in_size),
                             lambda c: (jnp.minimum(c, last), 0, 0)),
                pl.BlockSpec((in_size, hidden3), lambda c: (0, 0)),
                pl.BlockSpec((hidden, hidden3), lambda c: (0, 0)),
                pl.BlockSpec((1, hidden3), lambda c: (0, 0)),
                pl.BlockSpec((1, hidden), lambda c: (0, 0)),
                pl.BlockSpec((hidden, hidden3), lambda c: (0, 0)),
                pl.BlockSpec((hidden, hidden3), lambda c: (0, 0)),
                pl.BlockSpec((1, hidden3), lambda c: (0, 0)),
                pl.BlockSpec((1, hidden), lambda c: (0, 0)),
                pl.BlockSpec((hidden, fc1_out), lambda c: (0, 0)),
                pl.BlockSpec((1, fc1_out), lambda c: (0, 0)),
                pl.BlockSpec((fc1_out, num_classes), lambda c: (0, 0)),
                pl.BlockSpec((1, num_classes), lambda c: (0, 0)),
            ],
            out_specs=pl.BlockSpec(
                (t_chunk, b_pad, num_classes),
                lambda c: (jnp.maximum(c - 1, 0), 0, 0)),
            scratch_shapes=[
                pltpu.VMEM((t_chunk, b_pad, hidden3), jnp.bfloat16),
                pltpu.VMEM((t_chunk, b_pad, hidden3), jnp.bfloat16),
                pltpu.VMEM((t_chunk, b_pad, hidden), jnp.bfloat16),
                pltpu.VMEM((t_chunk, b_pad, hidden), jnp.bfloat16),
                pltpu.VMEM((b_pad, hidden), jnp.float32),
                pltpu.VMEM((b_pad, hidden), jnp.float32),
            ],
        ),
        compiler_params=pltpu.CompilerParams(
            dimension_semantics=("arbitrary",),
            vmem_limit_bytes=64 * 1024 * 1024,
        ),
    )(xp, l0_w_ih_t, l0_w_hh_t, l0_b_i, l0_b_hn,
      l1_w_ih_t, l1_w_hh_t, l1_b_i, l1_b_hn,
      fc1_w_t, fc1_b, fc2_w_t, fc2_b)

    if s_pad != seq or b_pad != batch:
        out = out[:seq, :batch]
    return out


# speedup vs baseline: 1.6861x; 1.0711x over previous
import jax
import jax.numpy as jnp
from jax import lax
from jax.experimental import pallas as pl
from jax.experimental.pallas import tpu as pltpu

_T_CHUNK = 32


def _round_up(v, m):
    return ((v + m - 1) // m) * m


def _fused_model_kernel(x_ref, wih0_ref, whh0_ref, bi0_ref, bhn0_ref,
                        wih1_ref, whh1_ref, bi1_ref, bhn1_ref,
                        w1_ref, b1_ref, w2_ref, b2_ref,
                        out_ref,
                        g0_scr, g1_scr, seq0_scr, seq1_scr, h0_scr, h1_scr):
    t_chunk, bb, in_size = x_ref.shape
    hidden = whh0_ref.shape[0]
    c = pl.program_id(0)
    last = pl.num_programs(0) - 1

    @pl.when(c == 0)
    def _():
        h0_scr[...] = jnp.zeros_like(h0_scr)
        h1_scr[...] = jnp.zeros_like(h1_scr)

    whh0 = whh0_ref[...]
    whh1 = whh1_ref[...]
    bhn0 = bhn0_ref[...].astype(jnp.float32)
    bhn1 = bhn1_ref[...].astype(jnp.float32)

    def gate_math(gi, gh, bhn, h):
        rz = jax.nn.sigmoid(gi[:, :2 * hidden] + gh[:, :2 * hidden])
        r = rz[:, :hidden]
        z = rz[:, hidden:]
        n = jnp.tanh(gi[:, 2 * hidden:] + r * (gh[:, 2 * hidden:] + bhn))
        return (1.0 - z) * n + z * h

    def step_l0(tt, h0):
        gh0 = jnp.dot(h0.astype(jnp.bfloat16), whh0,
                      preferred_element_type=jnp.float32)
        h0n = gate_math(g0_scr[tt].astype(jnp.float32), gh0, bhn0, h0)
        seq0_scr[tt] = h0n.astype(seq0_scr.dtype)
        return h0n

    def step_l1(tt, h1):
        gh1 = jnp.dot(h1.astype(jnp.bfloat16), whh1,
                      preferred_element_type=jnp.float32)
        h1n = gate_math(g1_scr[tt].astype(jnp.float32), gh1, bhn1, h1)
        seq1_scr[tt] = h1n.astype(seq1_scr.dtype)
        return h1n

    @pl.when(c < last)
    def _():
        xb = x_ref[...].reshape(t_chunk * bb, in_size).astype(jnp.bfloat16)
        g0 = jnp.dot(xb, wih0_ref[...], preferred_element_type=jnp.float32)
        g0_scr[...] = (g0 + bi0_ref[...]).reshape(g0_scr.shape).astype(
            g0_scr.dtype)

    @pl.when((c > 0) & (c < last))
    def _():
        def dual_step(tt, carry):
            h0, h1 = carry
            return (step_l0(tt, h0), step_l1(tt, h1))
        h0f, h1f = lax.fori_loop(0, t_chunk, dual_step,
                                 (h0_scr[...], h1_scr[...]), unroll=8)
        h0_scr[...] = h0f
        h1_scr[...] = h1f

    @pl.when(c == 0)
    def _():
        h0_scr[...] = lax.fori_loop(0, t_chunk, step_l0, h0_scr[...],
                                    unroll=8)

    @pl.when(c == last)
    def _():
        h1_scr[...] = lax.fori_loop(0, t_chunk, step_l1, h1_scr[...],
                                    unroll=8)

    @pl.when(c < last)
    def _():
        a0 = seq0_scr[...].reshape(t_chunk * bb, hidden)
        g1 = jnp.dot(a0, wih1_ref[...], preferred_element_type=jnp.float32)
        g1_scr[...] = (g1 + bi1_ref[...]).reshape(g1_scr.shape).astype(
            g1_scr.dtype)

    @pl.when(c > 0)
    def _():
        hs = jnp.maximum(seq1_scr[...].reshape(t_chunk * bb, hidden), 0)
        y = jnp.dot(hs.astype(jnp.bfloat16), w1_ref[...],
                    preferred_element_type=jnp.float32) + b1_ref[...]
        y = jnp.maximum(y, 0.0).astype(jnp.bfloat16)
        y = jnp.dot(y, w2_ref[...],
                    preferred_element_type=jnp.float32) + b2_ref[...]
        out_ref[...] = y.reshape(out_ref.shape).astype(out_ref.dtype)


def kernel(x, l0_w_ih_t, l0_w_hh_t, l0_b_i, l0_b_hn,
           l1_w_ih_t, l1_w_hh_t, l1_b_i, l1_b_hn,
           fc1_w_t, fc1_b, fc2_w_t, fc2_b):
    seq, batch, in_size = x.shape
    hidden = l0_w_hh_t.shape[0]
    hidden3 = l0_w_ih_t.shape[1]
    fc1_out = fc1_w_t.shape[1]
    num_classes = fc2_w_t.shape[1]

    t_chunk = min(_T_CHUNK, seq)
    b_pad = _round_up(batch, 8)
    s_pad = _round_up(seq, t_chunk)
    xp = x
    if b_pad != batch or s_pad != seq:
        xp = jnp.pad(x, ((0, s_pad - seq), (0, b_pad - batch), (0, 0)))

    num_chunks = s_pad // t_chunk
    last = num_chunks - 1

    def inv(shape):
        return pl.BlockSpec(shape, lambda c: tuple(0 for _ in shape),
                            pipeline_mode=pl.Buffered(1))

    out = pl.pallas_call(
        _fused_model_kernel,
        out_shape=jax.ShapeDtypeStruct((s_pad, b_pad, num_classes),
                                       jnp.float32),
        grid_spec=pltpu.PrefetchScalarGridSpec(
            num_scalar_prefetch=0,
            grid=(num_chunks + 1,),
            in_specs=[
                pl.BlockSpec((t_chunk, b_pad, in_size),
                             lambda c: (jnp.minimum(c, last), 0, 0)),
                inv((in_size, hidden3)),
                inv((hidden, hidden3)),
                inv((1, hidden3)),
                inv((1, hidden)),
                inv((hidden, hidden3)),
                inv((hidden, hidden3)),
                inv((1, hidden3)),
                inv((1, hidden)),
                inv((hidden, fc1_out)),
                inv((1, fc1_out)),
                inv((fc1_out, num_classes)),
                inv((1, num_classes)),
            ],
            out_specs=pl.BlockSpec(
                (t_chunk, b_pad, num_classes),
                lambda c: (jnp.maximum(c - 1, 0), 0, 0)),
            scratch_shapes=[
                pltpu.VMEM((t_chunk, b_pad, hidden3), jnp.bfloat16),
                pltpu.VMEM((t_chunk, b_pad, hidden3), jnp.bfloat16),
                pltpu.VMEM((t_chunk, b_pad, hidden), jnp.bfloat16),
                pltpu.VMEM((t_chunk, b_pad, hidden), jnp.bfloat16),
                pltpu.VMEM((b_pad, hidden), jnp.float32),
                pltpu.VMEM((b_pad, hidden), jnp.float32),
            ],
        ),
        compiler_params=pltpu.CompilerParams(
            dimension_semantics=("arbitrary",),
            vmem_limit_bytes=64 * 1024 * 1024,
        ),
    )(xp, l0_w_ih_t, l0_w_hh_t, l0_b_i, l0_b_hn,
      l1_w_ih_t, l1_w_hh_t, l1_b_i, l1_b_hn,
      fc1_w_t, fc1_b, fc2_w_t, fc2_b)

    if s_pad != seq or b_pad != batch:
        out = out[:seq, :batch]
    return out
